# baseline (device time: 9785 ns/iter reference)
import jax
import jax.numpy as jnp
from jax import lax
from jax.experimental import pallas as pl
from jax.experimental.pallas import tpu as pltpu

N_DEV = 8


def kernel(x, dy, gamma):
    m_per, d_model = x.shape

    def body(x_hbm, dy_hbm, out_ref,
             x_ref, dy_ref, accum_ref, gather_ref,
             copy_sems, send_sems, recv_sems):
        my = lax.axis_index("i")

        barrier_sem = pltpu.get_barrier_semaphore()
        for d in range(1, N_DEV):
            peer = lax.rem(my + d, N_DEV)
            pl.semaphore_signal(
                barrier_sem, inc=1,
                device_id=(peer,), device_id_type=pl.DeviceIdType.MESH,
            )

        cp_x = pltpu.make_async_copy(x_hbm, x_ref, copy_sems.at[0])
        cp_dy = pltpu.make_async_copy(dy_hbm, dy_ref, copy_sems.at[1])
        cp_x.start()
        cp_dy.start()

        cp_x.wait()
        xv = x_ref[...]
        cp_dy.wait()
        dyv = dy_ref[...]
        mu = jnp.mean(xv, axis=1, keepdims=True)
        var = jnp.mean(xv * xv, axis=1, keepdims=True) - mu * mu
        rstd = lax.rsqrt(var + 1e-5)
        xhat = (xv - mu) * rstd
        dgamma = jnp.sum(dyv * xhat, axis=0)
        dbeta = jnp.sum(dyv, axis=0)
        accum_ref[...] = jnp.stack([dgamma, dbeta])

        pl.semaphore_wait(barrier_sem, N_DEV - 1)

        rdmas = []
        for d in range(1, N_DEV):
            peer = lax.rem(my + d, N_DEV)
            rdma = pltpu.make_async_remote_copy(
                src_ref=accum_ref,
                dst_ref=gather_ref.at[d - 1],
                send_sem=send_sems.at[d - 1],
                recv_sem=recv_sems.at[d - 1],
                device_id=(peer,),
                device_id_type=pl.DeviceIdType.MESH,
            )
            rdma.start()
            rdmas.append(rdma)

        total = accum_ref[...]
        for d in range(1, N_DEV):
            rdmas[d - 1].wait_recv()
            total = total + gather_ref[d - 1]
        for d in range(1, N_DEV):
            rdmas[d - 1].wait_send()
        out_ref[...] = total

    return pl.pallas_call(
        body,
        out_shape=jax.ShapeDtypeStruct((2, d_model), jnp.float32),
        in_specs=[
            pl.BlockSpec(memory_space=pl.ANY),
            pl.BlockSpec(memory_space=pl.ANY),
        ],
        out_specs=pl.BlockSpec(memory_space=pltpu.VMEM),
        scratch_shapes=[
            pltpu.VMEM((m_per, d_model), jnp.float32),
            pltpu.VMEM((m_per, d_model), jnp.float32),
            pltpu.VMEM((2, d_model), jnp.float32),
            pltpu.VMEM((N_DEV - 1, 2, d_model), jnp.float32),
            pltpu.SemaphoreType.DMA((2,)),
            pltpu.SemaphoreType.DMA((N_DEV - 1,)),
            pltpu.SemaphoreType.DMA((N_DEV - 1,)),
        ],
        compiler_params=pltpu.CompilerParams(collective_id=0),
    )(x, dy)


# device time: 8580 ns/iter; 1.1404x vs baseline; 1.1404x over previous
import jax
import jax.numpy as jnp
from jax import lax
from jax.experimental import pallas as pl
from jax.experimental.pallas import tpu as pltpu

N_DEV = 8


def kernel(x, dy, gamma):
    m_per, d_model = x.shape
    x = pltpu.with_memory_space_constraint(x, pltpu.MemorySpace.HBM)
    dy = pltpu.with_memory_space_constraint(dy, pltpu.MemorySpace.HBM)

    def body(x_hbm, dy_hbm, out_ref,
             x_ref, dy_ref, accum_ref, gather_ref,
             copy_sems, send_sems, recv_sems):
        my = lax.axis_index("i")

        barrier_sem = pltpu.get_barrier_semaphore()
        for d in range(1, N_DEV):
            peer = lax.rem(my + d, N_DEV)
            pl.semaphore_signal(
                barrier_sem, inc=1,
                device_id=(peer,), device_id_type=pl.DeviceIdType.MESH,
            )

        cp_x = pltpu.make_async_copy(x_hbm, x_ref, copy_sems.at[0])
        cp_dy = pltpu.make_async_copy(dy_hbm, dy_ref, copy_sems.at[1])
        cp_x.start()
        cp_dy.start()

        cp_x.wait()
        xv = x_ref[...]
        cp_dy.wait()
        dyv = dy_ref[...]
        mu = jnp.mean(xv, axis=1, keepdims=True)
        var = jnp.mean(xv * xv, axis=1, keepdims=True) - mu * mu
        rstd = lax.rsqrt(var + 1e-5)
        xhat = (xv - mu) * rstd
        dgamma = jnp.sum(dyv * xhat, axis=0)
        dbeta = jnp.sum(dyv, axis=0)
        accum_ref[...] = jnp.stack([dgamma, dbeta])

        pl.semaphore_wait(barrier_sem, N_DEV - 1)

        rdmas = []
        for d in range(1, N_DEV):
            peer = lax.rem(my + d, N_DEV)
            rdma = pltpu.make_async_remote_copy(
                src_ref=accum_ref,
                dst_ref=gather_ref.at[d - 1],
                send_sem=send_sems.at[d - 1],
                recv_sem=recv_sems.at[d - 1],
                device_id=(peer,),
                device_id_type=pl.DeviceIdType.MESH,
            )
            rdma.start()
            rdmas.append(rdma)

        total = accum_ref[...]
        for d in range(1, N_DEV):
            rdmas[d - 1].wait_recv()
            total = total + gather_ref[d - 1]
        for d in range(1, N_DEV):
            rdmas[d - 1].wait_send()
        out_ref[...] = total

    return pl.pallas_call(
        body,
        out_shape=jax.ShapeDtypeStruct((2, d_model), jnp.float32),
        in_specs=[
            pl.BlockSpec(memory_space=pl.ANY),
            pl.BlockSpec(memory_space=pl.ANY),
        ],
        out_specs=pl.BlockSpec(memory_space=pltpu.VMEM),
        scratch_shapes=[
            pltpu.VMEM((m_per, d_model), jnp.float32),
            pltpu.VMEM((m_per, d_model), jnp.float32),
            pltpu.VMEM((2, d_model), jnp.float32),
            pltpu.VMEM((N_DEV - 1, 2, d_model), jnp.float32),
            pltpu.SemaphoreType.DMA((2,)),
            pltpu.SemaphoreType.DMA((N_DEV - 1,)),
            pltpu.SemaphoreType.DMA((N_DEV - 1,)),
        ],
        compiler_params=pltpu.CompilerParams(collective_id=0),
    )(x, dy)


# device time: 8532 ns/iter; 1.1469x vs baseline; 1.0056x over previous
import jax
import jax.numpy as jnp
from jax import lax
from jax.experimental import pallas as pl
from jax.experimental.pallas import tpu as pltpu

N_DEV = 8


def kernel(x, dy, gamma):
    m_per, d_model = x.shape
    x = pltpu.with_memory_space_constraint(x, pltpu.MemorySpace.HBM)
    dy = pltpu.with_memory_space_constraint(dy, pltpu.MemorySpace.HBM)

    def body(x_hbm, dy_hbm, out_ref,
             x_ref, dy_ref, accum_ref, gather_ref,
             copy_sems, send_sems, recv_sems):
        my = lax.axis_index("i")

        barrier_sem = pltpu.get_barrier_semaphore()
        for d in range(1, N_DEV):
            peer = lax.rem(my + d, N_DEV)
            pl.semaphore_signal(
                barrier_sem, inc=1,
                device_id=(peer,), device_id_type=pl.DeviceIdType.MESH,
            )

        cp_x = pltpu.make_async_copy(x_hbm, x_ref, copy_sems.at[0])
        cp_dy = pltpu.make_async_copy(dy_hbm, dy_ref, copy_sems.at[1])
        cp_x.start()
        cp_dy.start()

        cp_x.wait()
        xv = x_ref[...]
        mu = jnp.mean(xv, axis=1, keepdims=True)
        var = jnp.mean(xv * xv, axis=1, keepdims=True) - mu * mu
        rstd = lax.rsqrt(var + 1e-5)
        xhat = (xv - mu) * rstd
        cp_dy.wait()
        dyv = dy_ref[...]
        dgamma = jnp.sum(dyv * xhat, axis=0)
        dbeta = jnp.sum(dyv, axis=0)
        accum_ref[...] = jnp.stack([dgamma, dbeta])

        pl.semaphore_wait(barrier_sem, N_DEV - 1)

        rdmas = []
        for d in range(1, N_DEV):
            peer = lax.rem(my + d, N_DEV)
            rdma = pltpu.make_async_remote_copy(
                src_ref=accum_ref,
                dst_ref=gather_ref.at[d - 1],
                send_sem=send_sems.at[d - 1],
                recv_sem=recv_sems.at[d - 1],
                device_id=(peer,),
                device_id_type=pl.DeviceIdType.MESH,
            )
            rdma.start()
            rdmas.append(rdma)

        total = accum_ref[...]
        for d in range(1, N_DEV):
            rdmas[d - 1].wait_recv()
            total = total + gather_ref[d - 1]
        for d in range(1, N_DEV):
            rdmas[d - 1].wait_send()
        out_ref[...] = total

    return pl.pallas_call(
        body,
        out_shape=jax.ShapeDtypeStruct((2, d_model), jnp.float32),
        in_specs=[
            pl.BlockSpec(memory_space=pl.ANY),
            pl.BlockSpec(memory_space=pl.ANY),
        ],
        out_specs=pl.BlockSpec(memory_space=pltpu.VMEM),
        scratch_shapes=[
            pltpu.VMEM((m_per, d_model), jnp.float32),
            pltpu.VMEM((m_per, d_model), jnp.float32),
            pltpu.VMEM((2, d_model), jnp.float32),
            pltpu.VMEM((N_DEV - 1, 2, d_model), jnp.float32),
            pltpu.SemaphoreType.DMA((2,)),
            pltpu.SemaphoreType.DMA((N_DEV - 1,)),
            pltpu.SemaphoreType.DMA((N_DEV - 1,)),
        ],
        compiler_params=pltpu.CompilerParams(collective_id=0),
    )(x, dy)


# device time: 8527 ns/iter; 1.1475x vs baseline; 1.0006x over previous
import jax
import jax.numpy as jnp
from jax import lax
from jax.experimental import pallas as pl
from jax.experimental.pallas import tpu as pltpu

N_DEV = 8


def kernel(x, dy, gamma):
    m_per, d_model = x.shape
    x = pltpu.with_memory_space_constraint(x, pltpu.MemorySpace.HBM)
    dy = pltpu.with_memory_space_constraint(dy, pltpu.MemorySpace.HBM)

    def body(x_hbm, dy_hbm, out_ref,
             x_ref, dy_ref, accum_ref, gather_ref,
             copy_sems, send_sems, recv_sems):
        my = lax.axis_index("i")

        barrier_sem = pltpu.get_barrier_semaphore()
        for d in range(1, N_DEV):
            peer = lax.rem(my + d, N_DEV)
            pl.semaphore_signal(
                barrier_sem, inc=1,
                device_id=(peer,), device_id_type=pl.DeviceIdType.MESH,
            )

        h = m_per // 2
        cps = []
        for j in range(2):
            for i, (src, dst) in enumerate(
                [(x_hbm, x_ref), (dy_hbm, dy_ref)]
            ):
                cp = pltpu.make_async_copy(
                    src.at[pl.ds(j * h, h)],
                    dst.at[pl.ds(j * h, h)],
                    copy_sems.at[2 * j + i],
                )
                cp.start()
                cps.append(cp)
        cp_x0, cp_dy0, cp_x1, cp_dy1 = cps

        def half_partial(j):
            xv = x_ref[pl.ds(j * h, h), :]
            mu = jnp.mean(xv, axis=1, keepdims=True)
            var = jnp.mean(xv * xv, axis=1, keepdims=True) - mu * mu
            rstd = lax.rsqrt(var + 1e-5)
            xhat = (xv - mu) * rstd
            dyv = dy_ref[pl.ds(j * h, h), :]
            dgamma = jnp.sum(dyv * xhat, axis=0)
            dbeta = jnp.sum(dyv, axis=0)
            return jnp.stack([dgamma, dbeta])

        cp_x0.wait()
        cp_dy0.wait()
        p0 = half_partial(0)
        cp_x1.wait()
        cp_dy1.wait()
        accum_ref[...] = p0 + half_partial(1)

        pl.semaphore_wait(barrier_sem, N_DEV - 1)

        rdmas = []
        for d in range(1, N_DEV):
            peer = lax.rem(my + d, N_DEV)
            rdma = pltpu.make_async_remote_copy(
                src_ref=accum_ref,
                dst_ref=gather_ref.at[d - 1],
                send_sem=send_sems.at[d - 1],
                recv_sem=recv_sems.at[d - 1],
                device_id=(peer,),
                device_id_type=pl.DeviceIdType.MESH,
            )
            rdma.start()
            rdmas.append(rdma)

        total = accum_ref[...]
        for d in range(1, N_DEV):
            rdmas[d - 1].wait_recv()
            total = total + gather_ref[d - 1]
        for d in range(1, N_DEV):
            rdmas[d - 1].wait_send()
        out_ref[...] = total

    return pl.pallas_call(
        body,
        out_shape=jax.ShapeDtypeStruct((2, d_model), jnp.float32),
        in_specs=[
            pl.BlockSpec(memory_space=pl.ANY),
            pl.BlockSpec(memory_space=pl.ANY),
        ],
        out_specs=pl.BlockSpec(memory_space=pltpu.VMEM),
        scratch_shapes=[
            pltpu.VMEM((m_per, d_model), jnp.float32),
            pltpu.VMEM((m_per, d_model), jnp.float32),
            pltpu.VMEM((2, d_model), jnp.float32),
            pltpu.VMEM((N_DEV - 1, 2, d_model), jnp.float32),
            pltpu.SemaphoreType.DMA((4,)),
            pltpu.SemaphoreType.DMA((N_DEV - 1,)),
            pltpu.SemaphoreType.DMA((N_DEV - 1,)),
        ],
        compiler_params=pltpu.CompilerParams(collective_id=0),
    )(x, dy)


# device time: 8477 ns/iter; 1.1543x vs baseline; 1.0059x over previous
import jax
import jax.numpy as jnp
from jax import lax
from jax.experimental import pallas as pl
from jax.experimental.pallas import tpu as pltpu

N_DEV = 8


def kernel(x, dy, gamma):
    m_per, d_model = x.shape
    x = pltpu.with_memory_space_constraint(x, pltpu.MemorySpace.HBM)
    dy = pltpu.with_memory_space_constraint(dy, pltpu.MemorySpace.HBM)

    def body(x_hbm, dy_hbm, out_ref,
             x_ref, dy_ref, accum_ref, total_ref, gather_ref,
             copy_sems, send_sems, recv_sems):
        my = lax.axis_index("i")

        barrier_sem = pltpu.get_barrier_semaphore()
        for d in range(1, N_DEV):
            peer = lax.rem(my + d, N_DEV)
            pl.semaphore_signal(
                barrier_sem, inc=1,
                device_id=(peer,), device_id_type=pl.DeviceIdType.MESH,
            )

        h = m_per // 2
        cps = []
        for j in range(2):
            for i, (src, dst) in enumerate(
                [(x_hbm, x_ref), (dy_hbm, dy_ref)]
            ):
                cp = pltpu.make_async_copy(
                    src.at[pl.ds(j * h, h)],
                    dst.at[pl.ds(j * h, h)],
                    copy_sems.at[2 * j + i],
                )
                cp.start()
                cps.append(cp)
        cp_x0, cp_dy0, cp_x1, cp_dy1 = cps

        def half_partial(j):
            xv = x_ref[pl.ds(j * h, h), :]
            mu = jnp.mean(xv, axis=1, keepdims=True)
            var = jnp.mean(xv * xv, axis=1, keepdims=True) - mu * mu
            rstd = lax.rsqrt(var + 1e-5)
            xhat = (xv - mu) * rstd
            dyv = dy_ref[pl.ds(j * h, h), :]
            dgamma = jnp.sum(dyv * xhat, axis=0)
            dbeta = jnp.sum(dyv, axis=0)
            return jnp.stack([dgamma, dbeta])

        cp_x0.wait()
        cp_dy0.wait()
        p0 = half_partial(0)
        cp_x1.wait()
        cp_dy1.wait()
        accum_ref[...] = p0 + half_partial(1)

        pl.semaphore_wait(barrier_sem, N_DEV - 1)

        rdmas = []
        for d in range(1, N_DEV):
            peer = lax.rem(my + d, N_DEV)
            rdma = pltpu.make_async_remote_copy(
                src_ref=accum_ref,
                dst_ref=gather_ref.at[d - 1],
                send_sem=send_sems.at[d - 1],
                recv_sem=recv_sems.at[d - 1],
                device_id=(peer,),
                device_id_type=pl.DeviceIdType.MESH,
            )
            rdma.start()
            rdmas.append(rdma)

        total = accum_ref[...]
        for d in range(1, N_DEV):
            rdmas[d - 1].wait_recv()
            total = total + gather_ref[d - 1]
        total_ref[...] = total
        cp_out = pltpu.make_async_copy(total_ref, out_ref, copy_sems.at[0])
        cp_out.start()
        for d in range(1, N_DEV):
            rdmas[d - 1].wait_send()
        cp_out.wait()

    return pl.pallas_call(
        body,
        out_shape=jax.ShapeDtypeStruct((2, d_model), jnp.float32),
        in_specs=[
            pl.BlockSpec(memory_space=pl.ANY),
            pl.BlockSpec(memory_space=pl.ANY),
        ],
        out_specs=pl.BlockSpec(memory_space=pl.ANY),
        scratch_shapes=[
            pltpu.VMEM((m_per, d_model), jnp.float32),
            pltpu.VMEM((m_per, d_model), jnp.float32),
            pltpu.VMEM((2, d_model), jnp.float32),
            pltpu.VMEM((2, d_model), jnp.float32),
            pltpu.VMEM((N_DEV - 1, 2, d_model), jnp.float32),
            pltpu.SemaphoreType.DMA((4,)),
            pltpu.SemaphoreType.DMA((N_DEV - 1,)),
            pltpu.SemaphoreType.DMA((N_DEV - 1,)),
        ],
        compiler_params=pltpu.CompilerParams(collective_id=0),
    )(x, dy)
